# baseline (speedup 1.0000x reference)
import functools

import jax
import jax.numpy as jnp
from jax import lax
from jax.experimental import pallas as pl
from jax.experimental.pallas import tpu as pltpu
from jax.experimental.pallas import tpu_sc as plsc

N = 10000
E = 640000
NP = 10240
D_IN = 116
D_HID = 128
DH = 64
DO = 16

NC = 2
NS = 16
NW = NC * NS
CH = 125
ROWS_W = 160
ROWS_PER_TILE = NP // NS
WCH = 128
K2 = 2
K16 = 8
KDEG = 8
NBUF = 3
GAHEAD = 2
DEGWIN = 4
assert E == NW * ROWS_W * CH


@functools.cache
def _mesh():
  return plsc.VectorSubcoreMesh(
      core_axis_name="c", subcore_axis_name="s", num_cores=NC, num_subcores=NS
  )


def _fill(ref, value, nrows, ncols):
  @pl.loop(0, nrows)
  def _(r):
    for j in range(ncols // 16):
      ref[r, pl.ds(j * 16, 16)] = jnp.full((16,), value, jnp.float32)


def _zero_acc_slice(acc, zbuf, base):
  for k in range(ROWS_PER_TILE // WCH):
    pltpu.sync_copy(zbuf, acc.at[pl.ds(base + k * WCH, WCH)])


def _writeout(acc, out_ref, rows, sem, base):
  for k in range(ROWS_PER_TILE // WCH):
    r0 = base + k * WCH
    pltpu.sync_copy(acc.at[pl.ds(r0, WCH)], rows)
    pltpu.async_copy(rows, out_ref.at[pl.ds(r0, WCH)], sem).wait()


@functools.cache
def _make_deg_kernel():
  steps = ROWS_W // KDEG

  def body(dst_hbm, out_hbm, didx, ones, rows, acc, sem_s):
    cid = lax.axis_index("c")
    sid = lax.axis_index("s")
    wid = sid * NC + cid
    base = sid * ROWS_PER_TILE

    pltpu.sync_copy(dst_hbm.at[pl.ds(wid * steps, steps)], didx)
    _fill(rows, 0.0, WCH, DO)
    _zero_acc_slice(acc, rows, base)
    _fill(ones, 1.0, KDEG * CH, DO)
    plsc.subcore_barrier()

    for m in range(DEGWIN):
      pltpu.async_copy(ones, acc.at[didx.at[m]], sem_s, add=True)

    @pl.loop(0, steps - DEGWIN)
    def _(m):
      pltpu.make_async_copy(ones, acc.at[didx.at[0]], sem_s).wait()
      pltpu.async_copy(ones, acc.at[didx.at[m + DEGWIN]], sem_s, add=True)

    for m in range(DEGWIN):
      pltpu.make_async_copy(ones, acc.at[didx.at[0]], sem_s).wait()

    plsc.subcore_barrier()
    _writeout(acc, out_hbm.at[cid], rows, sem_s, base)

  return pl.kernel(
      body,
      out_type=jax.ShapeDtypeStruct((NC, NP, DO), jnp.float32),
      mesh=_mesh(),
      compiler_params=pltpu.CompilerParams(use_tc_tiling_on_sc=False),
      scratch_types=[
          pltpu.VMEM((ROWS_W // KDEG, KDEG * CH), jnp.int32),
          pltpu.VMEM((KDEG * CH, DO), jnp.float32),
          pltpu.VMEM((WCH, DO), jnp.float32),
          pltpu.VMEM_SHARED((NP, DO), jnp.float32),
          pltpu.SemaphoreType.DMA,
      ],
  )


def _pipeline_phase(g_hbm, sidx, didx, rows, acc, sem_g, sem_s, k):
  steps = ROWS_W // k
  main = (steps // NBUF) * NBUF

  def _wait_gather(b):
    pltpu.make_async_copy(g_hbm.at[sidx.at[0]], rows[b], sem_g).wait()

  def _wait_scatter(b):
    pltpu.make_async_copy(rows[b], acc.at[didx.at[0]], sem_s).wait()

  for b in range(GAHEAD):
    pltpu.async_copy(g_hbm.at[sidx.at[b]], rows[b], sem_g)

  @pl.loop(0, steps // NBUF)
  def _(t):
    for b in range(NBUF):
      m = t * NBUF + b
      _wait_gather(b)
      pltpu.async_copy(rows[b], acc.at[didx.at[m]], sem_s, add=True)
      @pl.when(m >= NBUF - GAHEAD)
      def _():
        _wait_scatter(b)
      @pl.when(m + GAHEAD < steps)
      def _():
        b2 = (b + GAHEAD) % NBUF
        pltpu.async_copy(g_hbm.at[sidx.at[m + GAHEAD]], rows[b2], sem_g)

  for m in range(main, steps):
    b = m % NBUF
    _wait_gather(b)
    pltpu.async_copy(rows[b], acc.at[didx.at[m]], sem_s, add=True)
    if m >= NBUF - GAHEAD:
      _wait_scatter(b)
    if m + GAHEAD < steps:
      pltpu.async_copy(g_hbm.at[sidx.at[m + GAHEAD]],
                       rows[(m + GAHEAD) % NBUF], sem_g)

  for _i in range(NBUF - GAHEAD):
    _wait_scatter(0)


@functools.cache
def _make_prop2_kernel():
  def body(glo_hbm, ghi_hbm, src_hbm, dst_hbm, out_hbm,
           sidx, didx, rows, acc, sem_g, sem_s):
    cid = lax.axis_index("c")
    sid = lax.axis_index("s")
    wid = sid * NC + cid
    base = sid * ROWS_PER_TILE

    nst = ROWS_W // K2
    pltpu.sync_copy(src_hbm.at[pl.ds(wid * nst, nst)], sidx)
    pltpu.sync_copy(dst_hbm.at[pl.ds(wid * nst, nst)], didx)

    for p, g_hbm in enumerate((glo_hbm, ghi_hbm)):
      _fill(rows[0], 0.0, WCH, DH)
      _zero_acc_slice(acc, rows[0].at[pl.ds(0, WCH)], base)
      plsc.subcore_barrier()
      _pipeline_phase(g_hbm, sidx, didx, rows, acc, sem_g, sem_s, K2)
      plsc.subcore_barrier()
      _writeout(acc, out_hbm.at[cid, p], rows[0].at[pl.ds(0, WCH)], sem_s, base)
      if p == 0:
        plsc.subcore_barrier()

  return pl.kernel(
      body,
      out_type=jax.ShapeDtypeStruct((NC, 2, NP, DH), jnp.float32),
      mesh=_mesh(),
      compiler_params=pltpu.CompilerParams(use_tc_tiling_on_sc=False),
      scratch_types=[
          pltpu.VMEM((ROWS_W // K2, K2 * CH), jnp.int32),
          pltpu.VMEM((ROWS_W // K2, K2 * CH), jnp.int32),
          [pltpu.VMEM((K2 * CH, DH), jnp.float32)] * NBUF,
          pltpu.VMEM_SHARED((NP, DH), jnp.float32),
          pltpu.SemaphoreType.DMA,
          pltpu.SemaphoreType.DMA,
      ],
  )


@functools.cache
def _make_prop16_kernel():
  def body(g_hbm, src_hbm, dst_hbm, out_hbm,
           sidx, didx, rows, acc, sem_g, sem_s):
    cid = lax.axis_index("c")
    sid = lax.axis_index("s")
    wid = sid * NC + cid
    base = sid * ROWS_PER_TILE

    nst = ROWS_W // K16
    pltpu.sync_copy(src_hbm.at[pl.ds(wid * nst, nst)], sidx)
    pltpu.sync_copy(dst_hbm.at[pl.ds(wid * nst, nst)], didx)
    _fill(rows[0], 0.0, WCH, DO)
    _zero_acc_slice(acc, rows[0].at[pl.ds(0, WCH)], base)
    plsc.subcore_barrier()
    _pipeline_phase(g_hbm, sidx, didx, rows, acc, sem_g, sem_s, K16)
    plsc.subcore_barrier()
    _writeout(acc, out_hbm.at[cid], rows[0].at[pl.ds(0, WCH)], sem_s, base)

  return pl.kernel(
      body,
      out_type=jax.ShapeDtypeStruct((NC, NP, DO), jnp.float32),
      mesh=_mesh(),
      compiler_params=pltpu.CompilerParams(use_tc_tiling_on_sc=False),
      scratch_types=[
          pltpu.VMEM((ROWS_W // K16, K16 * CH), jnp.int32),
          pltpu.VMEM((ROWS_W // K16, K16 * CH), jnp.int32),
          [pltpu.VMEM((K16 * CH, DO), jnp.float32)] * NBUF,
          pltpu.VMEM_SHARED((NP, DO), jnp.float32),
          pltpu.SemaphoreType.DMA,
          pltpu.SemaphoreType.DMA,
      ],
  )


_BR = 1024


def _tc_first(degp, x, w):
  def body(dp_ref, x_ref, w_ref, dinv_ref, glo_ref, ghi_ref):
    dv = lax.rsqrt(dp_ref[0, :, 0:1] + dp_ref[1, :, 0:1] + 1.0)
    dinv_ref[...] = dv
    g = dv * jnp.dot(x_ref[...], w_ref[...],
                     preferred_element_type=jnp.float32)
    glo_ref[...] = g[:, :DH]
    ghi_ref[...] = g[:, DH:]

  return pl.pallas_call(
      body,
      grid=(NP // _BR,),
      in_specs=[
          pl.BlockSpec((NC, _BR, DO), lambda i: (0, i, 0)),
          pl.BlockSpec((_BR, D_HID), lambda i: (i, 0)),
          pl.BlockSpec((D_HID, D_HID), lambda i: (0, 0)),
      ],
      out_specs=[
          pl.BlockSpec((_BR, 1), lambda i: (i, 0)),
          pl.BlockSpec((_BR, DH), lambda i: (i, 0)),
          pl.BlockSpec((_BR, DH), lambda i: (i, 0)),
      ],
      out_shape=[
          jax.ShapeDtypeStruct((NP, 1), jnp.float32),
          jax.ShapeDtypeStruct((NP, DH), jnp.float32),
          jax.ShapeDtypeStruct((NP, DH), jnp.float32),
      ],
  )(degp, x, w)


def _tc_combine(dinv, parts, glo, ghi, b, w, split_out):
  width_out = D_HID if split_out else DO

  def body(dv_ref, p_ref, glo_ref, ghi_ref, b_ref, w_ref, *out_refs):
    dv = dv_ref[...]
    h = jnp.concatenate(
        [p_ref[0, 0] + p_ref[1, 0] + glo_ref[...],
         p_ref[0, 1] + p_ref[1, 1] + ghi_ref[...]], axis=1)
    y = jnp.maximum(dv * h + b_ref[...], 0.0)
    g = dv * jnp.dot(y, w_ref[...], preferred_element_type=jnp.float32)
    if split_out:
      out_refs[0][...] = g[:, :DH]
      out_refs[1][...] = g[:, DH:]
    else:
      out_refs[0][...] = g

  if split_out:
    out_specs = [pl.BlockSpec((_BR, DH), lambda i: (i, 0))] * 2
    out_shape = [jax.ShapeDtypeStruct((NP, DH), jnp.float32)] * 2
  else:
    out_specs = [pl.BlockSpec((_BR, DO), lambda i: (i, 0))]
    out_shape = [jax.ShapeDtypeStruct((NP, DO), jnp.float32)]

  return pl.pallas_call(
      body,
      grid=(NP // _BR,),
      in_specs=[
          pl.BlockSpec((_BR, 1), lambda i: (i, 0)),
          pl.BlockSpec((NC, 2, _BR, DH), lambda i: (0, 0, i, 0)),
          pl.BlockSpec((_BR, DH), lambda i: (i, 0)),
          pl.BlockSpec((_BR, DH), lambda i: (i, 0)),
          pl.BlockSpec((1, D_HID), lambda i: (0, 0)),
          pl.BlockSpec((D_HID, width_out), lambda i: (0, 0)),
      ],
      out_specs=out_specs,
      out_shape=out_shape,
  )(dinv, parts, glo, ghi, b, w)


def _tc_final(dinv, parts, g, b, d_out):
  brf = 1000

  def body(dv_ref, p_ref, g_ref, b_ref, out_ref):
    v = dv_ref[...] * (p_ref[0] + p_ref[1] + g_ref[...]) + b_ref[...]
    out_ref[...] = v[:, :d_out]

  return pl.pallas_call(
      body,
      grid=(N // brf,),
      in_specs=[
          pl.BlockSpec((brf, 1), lambda i: (i, 0)),
          pl.BlockSpec((NC, brf, DO), lambda i: (0, i, 0)),
          pl.BlockSpec((brf, DO), lambda i: (i, 0)),
          pl.BlockSpec((1, DO), lambda i: (0, 0)),
      ],
      out_specs=pl.BlockSpec((brf, d_out), lambda i: (i, 0)),
      out_shape=jax.ShapeDtypeStruct((N, d_out), jnp.float32),
  )(dinv, parts, g, b)


@jax.jit
def kernel(x, edge_index, W1, b1, W2, b2, W3, b3):
  _deg_kernel = _make_deg_kernel()
  _prop2 = _make_prop2_kernel()
  _prop16 = _make_prop16_kernel()

  src2 = edge_index[0].reshape(NW * (ROWS_W // K2), K2 * CH)
  dst2 = edge_index[1].reshape(NW * (ROWS_W // K2), K2 * CH)
  src16 = edge_index[0].reshape(NW * (ROWS_W // K16), K16 * CH)
  dst16 = edge_index[1].reshape(NW * (ROWS_W // K16), K16 * CH)

  x_pad = jnp.zeros((NP, D_HID), jnp.float32).at[:N, :D_IN].set(x)
  w1p = jnp.zeros((D_HID, D_HID), jnp.float32).at[:D_IN].set(W1)
  w3p = jnp.zeros((D_HID, DO), jnp.float32).at[:, : W3.shape[1]].set(W3)
  b1r = b1.reshape(1, D_HID)
  b2r = b2.reshape(1, D_HID)
  b3r = jnp.zeros((1, DO), jnp.float32).at[0, : b3.shape[0]].set(b3)

  degp = _deg_kernel(dst16)
  dinv, g1lo, g1hi = _tc_first(degp, x_pad, w1p)
  p1 = _prop2(g1lo, g1hi, src2, dst2)
  g2lo, g2hi = _tc_combine(dinv, p1, g1lo, g1hi, b1r, W2, split_out=True)
  p2 = _prop2(g2lo, g2hi, src2, dst2)
  (g3,) = _tc_combine(dinv, p2, g2lo, g2hi, b2r, w3p, split_out=False)
  p3 = _prop16(g3, src16, dst16)
  return _tc_final(dinv, p3, g3, b3r, W3.shape[1])


# speedup vs baseline: 1.0430x; 1.0430x over previous
import functools

import jax
import jax.numpy as jnp
from jax import lax
from jax.experimental import pallas as pl
from jax.experimental.pallas import tpu as pltpu
from jax.experimental.pallas import tpu_sc as plsc

N = 10000
E = 640000
NP = 10240
D_IN = 116
D_HID = 128
DH = 64
DO = 16

NC = 2
NS = 16
NW = NC * NS
CH = 120
ROWS_W = 168
E_PAD = NW * ROWS_W * CH
ROWS_PER_TILE = NP // NS
WCH = 128
K2 = 2
K16 = 8
KDEG = 8
NBUF = 3
GAHEAD = 2
DEGWIN = 4


@functools.cache
def _mesh():
  return plsc.VectorSubcoreMesh(
      core_axis_name="c", subcore_axis_name="s", num_cores=NC, num_subcores=NS
  )


def _fill(ref, value, nrows, ncols):
  @pl.loop(0, nrows)
  def _(r):
    for j in range(ncols // 16):
      ref[r, pl.ds(j * 16, 16)] = jnp.full((16,), value, jnp.float32)


def _zero_acc_slice(acc, zbuf, base):
  for k in range(ROWS_PER_TILE // WCH):
    pltpu.sync_copy(zbuf, acc.at[pl.ds(base + k * WCH, WCH)])


def _writeout(acc, out_ref, rows, sem, base):
  for k in range(ROWS_PER_TILE // WCH):
    r0 = base + k * WCH
    pltpu.sync_copy(acc.at[pl.ds(r0, WCH)], rows)
    pltpu.async_copy(rows, out_ref.at[pl.ds(r0, WCH)], sem).wait()


@functools.cache
def _make_deg_kernel():
  steps = ROWS_W // KDEG

  def body(dst_hbm, out_hbm, didx, ones, rows, acc, sem_s):
    cid = lax.axis_index("c")
    sid = lax.axis_index("s")
    wid = sid * NC + cid
    base = sid * ROWS_PER_TILE

    pltpu.sync_copy(dst_hbm.at[pl.ds(wid * steps, steps)], didx)
    _fill(rows, 0.0, WCH, DO)
    _zero_acc_slice(acc, rows, base)
    _fill(ones, 1.0, KDEG * CH, DO)
    plsc.subcore_barrier()

    for m in range(DEGWIN):
      pltpu.async_copy(ones, acc.at[didx.at[m]], sem_s, add=True)

    @pl.loop(0, steps - DEGWIN)
    def _(m):
      pltpu.make_async_copy(ones, acc.at[didx.at[0]], sem_s).wait()
      pltpu.async_copy(ones, acc.at[didx.at[m + DEGWIN]], sem_s, add=True)

    for m in range(DEGWIN):
      pltpu.make_async_copy(ones, acc.at[didx.at[0]], sem_s).wait()

    plsc.subcore_barrier()
    _writeout(acc, out_hbm.at[cid], rows, sem_s, base)

  return pl.kernel(
      body,
      out_type=jax.ShapeDtypeStruct((NC, NP, DO), jnp.float32),
      mesh=_mesh(),
      compiler_params=pltpu.CompilerParams(use_tc_tiling_on_sc=False),
      scratch_types=[
          pltpu.VMEM((ROWS_W // KDEG, KDEG * CH), jnp.int32),
          pltpu.VMEM((KDEG * CH, DO), jnp.float32),
          pltpu.VMEM((WCH, DO), jnp.float32),
          pltpu.VMEM_SHARED((NP, DO), jnp.float32),
          pltpu.SemaphoreType.DMA,
      ],
  )


def _pipeline_phase(g_hbm, sidx, didx, rows, acc, sem_g, sem_s, k):
  steps = ROWS_W // k
  main = (steps // NBUF) * NBUF

  def _wait_gather(b):
    pltpu.make_async_copy(g_hbm.at[sidx.at[0]], rows[b], sem_g).wait()

  def _wait_scatter(b):
    pltpu.make_async_copy(rows[b], acc.at[didx.at[0]], sem_s).wait()

  for b in range(GAHEAD):
    pltpu.async_copy(g_hbm.at[sidx.at[b]], rows[b], sem_g)

  @pl.loop(0, steps // NBUF)
  def _(t):
    for b in range(NBUF):
      m = t * NBUF + b
      _wait_gather(b)
      pltpu.async_copy(rows[b], acc.at[didx.at[m]], sem_s, add=True)
      @pl.when(m >= NBUF - GAHEAD)
      def _():
        _wait_scatter(b)
      @pl.when(m + GAHEAD < steps)
      def _():
        b2 = (b + GAHEAD) % NBUF
        pltpu.async_copy(g_hbm.at[sidx.at[m + GAHEAD]], rows[b2], sem_g)

  for m in range(main, steps):
    b = m % NBUF
    _wait_gather(b)
    pltpu.async_copy(rows[b], acc.at[didx.at[m]], sem_s, add=True)
    if m >= NBUF - GAHEAD:
      _wait_scatter(b)
    if m + GAHEAD < steps:
      pltpu.async_copy(g_hbm.at[sidx.at[m + GAHEAD]],
                       rows[(m + GAHEAD) % NBUF], sem_g)

  for _i in range(NBUF - GAHEAD):
    _wait_scatter(0)


@functools.cache
def _make_prop2_kernel():
  def body(glo_hbm, ghi_hbm, src_hbm, dst_hbm, out_hbm,
           sidx, didx, rows, acc, sem_g, sem_s):
    cid = lax.axis_index("c")
    sid = lax.axis_index("s")
    wid = sid * NC + cid
    base = sid * ROWS_PER_TILE

    nst = ROWS_W // K2
    pltpu.sync_copy(src_hbm.at[pl.ds(wid * nst, nst)], sidx)
    pltpu.sync_copy(dst_hbm.at[pl.ds(wid * nst, nst)], didx)

    for p, g_hbm in enumerate((glo_hbm, ghi_hbm)):
      _fill(rows[0], 0.0, WCH, DH)
      _zero_acc_slice(acc, rows[0].at[pl.ds(0, WCH)], base)
      plsc.subcore_barrier()
      _pipeline_phase(g_hbm, sidx, didx, rows, acc, sem_g, sem_s, K2)
      plsc.subcore_barrier()
      _writeout(acc, out_hbm.at[cid, p], rows[0].at[pl.ds(0, WCH)], sem_s, base)
      if p == 0:
        plsc.subcore_barrier()

  return pl.kernel(
      body,
      out_type=jax.ShapeDtypeStruct((NC, 2, NP, DH), jnp.float32),
      mesh=_mesh(),
      compiler_params=pltpu.CompilerParams(use_tc_tiling_on_sc=False),
      scratch_types=[
          pltpu.VMEM((ROWS_W // K2, K2 * CH), jnp.int32),
          pltpu.VMEM((ROWS_W // K2, K2 * CH), jnp.int32),
          [pltpu.VMEM((K2 * CH, DH), jnp.float32)] * NBUF,
          pltpu.VMEM_SHARED((NP, DH), jnp.float32),
          pltpu.SemaphoreType.DMA,
          pltpu.SemaphoreType.DMA,
      ],
  )


@functools.cache
def _make_prop16_kernel():
  def body(g_hbm, src_hbm, dst_hbm, out_hbm,
           sidx, didx, rows, acc, sem_g, sem_s):
    cid = lax.axis_index("c")
    sid = lax.axis_index("s")
    wid = sid * NC + cid
    base = sid * ROWS_PER_TILE

    nst = ROWS_W // K16
    pltpu.sync_copy(src_hbm.at[pl.ds(wid * nst, nst)], sidx)
    pltpu.sync_copy(dst_hbm.at[pl.ds(wid * nst, nst)], didx)
    _fill(rows[0], 0.0, WCH, DO)
    _zero_acc_slice(acc, rows[0].at[pl.ds(0, WCH)], base)
    plsc.subcore_barrier()
    _pipeline_phase(g_hbm, sidx, didx, rows, acc, sem_g, sem_s, K16)
    plsc.subcore_barrier()
    _writeout(acc, out_hbm.at[cid], rows[0].at[pl.ds(0, WCH)], sem_s, base)

  return pl.kernel(
      body,
      out_type=jax.ShapeDtypeStruct((NC, NP, DO), jnp.float32),
      mesh=_mesh(),
      compiler_params=pltpu.CompilerParams(use_tc_tiling_on_sc=False),
      scratch_types=[
          pltpu.VMEM((ROWS_W // K16, K16 * CH), jnp.int32),
          pltpu.VMEM((ROWS_W // K16, K16 * CH), jnp.int32),
          [pltpu.VMEM((K16 * CH, DO), jnp.float32)] * NBUF,
          pltpu.VMEM_SHARED((NP, DO), jnp.float32),
          pltpu.SemaphoreType.DMA,
          pltpu.SemaphoreType.DMA,
      ],
  )


_BR = 1024


def _tc_first(degp, x, w):
  def body(dp_ref, x_ref, w_ref, dinv_ref, glo_ref, ghi_ref):
    dv = lax.rsqrt(dp_ref[0, :, 0:1] + dp_ref[1, :, 0:1] + 1.0)
    dinv_ref[...] = dv
    g = dv * jnp.dot(x_ref[...], w_ref[...],
                     preferred_element_type=jnp.float32)
    glo_ref[...] = g[:, :DH]
    ghi_ref[...] = g[:, DH:]

  return pl.pallas_call(
      body,
      grid=(NP // _BR,),
      in_specs=[
          pl.BlockSpec((NC, _BR, DO), lambda i: (0, i, 0)),
          pl.BlockSpec((_BR, D_HID), lambda i: (i, 0)),
          pl.BlockSpec((D_HID, D_HID), lambda i: (0, 0)),
      ],
      out_specs=[
          pl.BlockSpec((_BR, 1), lambda i: (i, 0)),
          pl.BlockSpec((_BR, DH), lambda i: (i, 0)),
          pl.BlockSpec((_BR, DH), lambda i: (i, 0)),
      ],
      out_shape=[
          jax.ShapeDtypeStruct((NP, 1), jnp.float32),
          jax.ShapeDtypeStruct((NP, DH), jnp.float32),
          jax.ShapeDtypeStruct((NP, DH), jnp.float32),
      ],
  )(degp, x, w)


def _tc_combine(dinv, parts, glo, ghi, b, w, split_out):
  width_out = D_HID if split_out else DO

  def body(dv_ref, p_ref, glo_ref, ghi_ref, b_ref, w_ref, *out_refs):
    dv = dv_ref[...]
    h = jnp.concatenate(
        [p_ref[0, 0] + p_ref[1, 0] + glo_ref[...],
         p_ref[0, 1] + p_ref[1, 1] + ghi_ref[...]], axis=1)
    y = jnp.maximum(dv * h + b_ref[...], 0.0)
    g = dv * jnp.dot(y, w_ref[...], preferred_element_type=jnp.float32)
    if split_out:
      out_refs[0][...] = g[:, :DH]
      out_refs[1][...] = g[:, DH:]
    else:
      out_refs[0][...] = g

  if split_out:
    out_specs = [pl.BlockSpec((_BR, DH), lambda i: (i, 0))] * 2
    out_shape = [jax.ShapeDtypeStruct((NP, DH), jnp.float32)] * 2
  else:
    out_specs = [pl.BlockSpec((_BR, DO), lambda i: (i, 0))]
    out_shape = [jax.ShapeDtypeStruct((NP, DO), jnp.float32)]

  return pl.pallas_call(
      body,
      grid=(NP // _BR,),
      in_specs=[
          pl.BlockSpec((_BR, 1), lambda i: (i, 0)),
          pl.BlockSpec((NC, 2, _BR, DH), lambda i: (0, 0, i, 0)),
          pl.BlockSpec((_BR, DH), lambda i: (i, 0)),
          pl.BlockSpec((_BR, DH), lambda i: (i, 0)),
          pl.BlockSpec((1, D_HID), lambda i: (0, 0)),
          pl.BlockSpec((D_HID, width_out), lambda i: (0, 0)),
      ],
      out_specs=out_specs,
      out_shape=out_shape,
  )(dinv, parts, glo, ghi, b, w)


def _tc_final(dinv, parts, g, b, d_out):
  brf = 1000

  def body(dv_ref, p_ref, g_ref, b_ref, out_ref):
    v = dv_ref[...] * (p_ref[0] + p_ref[1] + g_ref[...]) + b_ref[...]
    out_ref[...] = v[:, :d_out]

  return pl.pallas_call(
      body,
      grid=(N // brf,),
      in_specs=[
          pl.BlockSpec((brf, 1), lambda i: (i, 0)),
          pl.BlockSpec((NC, brf, DO), lambda i: (0, i, 0)),
          pl.BlockSpec((brf, DO), lambda i: (i, 0)),
          pl.BlockSpec((1, DO), lambda i: (0, 0)),
      ],
      out_specs=pl.BlockSpec((brf, d_out), lambda i: (i, 0)),
      out_shape=jax.ShapeDtypeStruct((N, d_out), jnp.float32),
  )(dinv, parts, g, b)


@jax.jit
def kernel(x, edge_index, W1, b1, W2, b2, W3, b3):
  _deg_kernel = _make_deg_kernel()
  _prop2 = _make_prop2_kernel()
  _prop16 = _make_prop16_kernel()

  pad = (N + (jnp.arange(E_PAD - E, dtype=jnp.int32) % (NP - N))).astype(
      jnp.int32)
  src_flat = jnp.concatenate([edge_index[0], pad])
  dst_flat = jnp.concatenate([edge_index[1], pad])
  src2 = src_flat.reshape(NW * (ROWS_W // K2), K2 * CH)
  dst2 = dst_flat.reshape(NW * (ROWS_W // K2), K2 * CH)
  src16 = src_flat.reshape(NW * (ROWS_W // K16), K16 * CH)
  dst16 = dst_flat.reshape(NW * (ROWS_W // K16), K16 * CH)

  x_pad = jnp.zeros((NP, D_HID), jnp.float32).at[:N, :D_IN].set(x)
  w1p = jnp.zeros((D_HID, D_HID), jnp.float32).at[:D_IN].set(W1)
  w3p = jnp.zeros((D_HID, DO), jnp.float32).at[:, : W3.shape[1]].set(W3)
  b1r = b1.reshape(1, D_HID)
  b2r = b2.reshape(1, D_HID)
  b3r = jnp.zeros((1, DO), jnp.float32).at[0, : b3.shape[0]].set(b3)

  degp = _deg_kernel(dst16)
  dinv, g1lo, g1hi = _tc_first(degp, x_pad, w1p)
  p1 = _prop2(g1lo, g1hi, src2, dst2)
  g2lo, g2hi = _tc_combine(dinv, p1, g1lo, g1hi, b1r, W2, split_out=True)
  p2 = _prop2(g2lo, g2hi, src2, dst2)
  (g3,) = _tc_combine(dinv, p2, g2lo, g2hi, b2r, w3p, split_out=False)
  p3 = _prop16(g3, src16, dst16)
  return _tc_final(dinv, p3, g3, b3r, W3.shape[1])


# speedup vs baseline: 1.1184x; 1.0723x over previous
import functools

import jax
import jax.numpy as jnp
from jax import lax
from jax.experimental import pallas as pl
from jax.experimental.pallas import tpu as pltpu
from jax.experimental.pallas import tpu_sc as plsc

N = 10000
E = 640000
NP = 10240
D_IN = 116
D_HID = 128
DH = 64
DO = 16

NC = 2
NS = 16
NW = NC * NS
CH = 120
ROWS_W = 168
E_PAD = NW * ROWS_W * CH
ROWS_PER_TILE = NP // NS
WCH = 128
K2 = 2
K16 = 8
KDEG = 8
NBUF = 3
GAHEAD = 2
DEGWIN = 4


@functools.cache
def _mesh():
  return plsc.VectorSubcoreMesh(
      core_axis_name="c", subcore_axis_name="s", num_cores=NC, num_subcores=NS
  )


def _fill(ref, value, nrows, ncols):
  @pl.loop(0, nrows)
  def _(r):
    for j in range(ncols // 16):
      ref[r, pl.ds(j * 16, 16)] = jnp.full((16,), value, jnp.float32)


def _zero_acc_slice(acc, zbuf, base):
  for k in range(ROWS_PER_TILE // WCH):
    pltpu.sync_copy(zbuf, acc.at[pl.ds(base + k * WCH, WCH)])


def _writeout(acc, out_ref, rows, sem, base):
  for k in range(ROWS_PER_TILE // WCH):
    r0 = base + k * WCH
    pltpu.sync_copy(acc.at[pl.ds(r0, WCH)], rows)
    pltpu.async_copy(rows, out_ref.at[pl.ds(r0, WCH)], sem).wait()


@functools.cache
def _make_deg_kernel():
  steps = ROWS_W // KDEG

  def body(dst_hbm, out_hbm, didx, ones, rows, acc, sem_s):
    cid = lax.axis_index("c")
    sid = lax.axis_index("s")
    wid = sid * NC + cid
    base = sid * ROWS_PER_TILE

    pltpu.sync_copy(dst_hbm.at[pl.ds(wid * steps, steps)], didx)
    _fill(rows, 0.0, WCH, DO)
    _zero_acc_slice(acc, rows, base)
    _fill(ones, 1.0, KDEG * CH, DO)
    plsc.subcore_barrier()

    for m in range(DEGWIN):
      pltpu.async_copy(ones, acc.at[didx.at[m]], sem_s, add=True)

    @pl.loop(0, steps - DEGWIN)
    def _(m):
      pltpu.make_async_copy(ones, acc.at[didx.at[0]], sem_s).wait()
      pltpu.async_copy(ones, acc.at[didx.at[m + DEGWIN]], sem_s, add=True)

    for m in range(DEGWIN):
      pltpu.make_async_copy(ones, acc.at[didx.at[0]], sem_s).wait()

    plsc.subcore_barrier()
    _writeout(acc, out_hbm.at[cid], rows, sem_s, base)

  return pl.kernel(
      body,
      out_type=jax.ShapeDtypeStruct((NC, NP, DO), jnp.float32),
      mesh=_mesh(),
      compiler_params=pltpu.CompilerParams(use_tc_tiling_on_sc=False),
      scratch_types=[
          pltpu.VMEM((ROWS_W // KDEG, KDEG * CH), jnp.int32),
          pltpu.VMEM((KDEG * CH, DO), jnp.float32),
          pltpu.VMEM((WCH, DO), jnp.float32),
          pltpu.VMEM_SHARED((NP, DO), jnp.float32),
          pltpu.SemaphoreType.DMA,
      ],
  )


def _pipeline_phase(g_hbm, sidx, didx, rows, acc, sem_g, sem_s, k):
  steps = ROWS_W // k
  main = (steps // NBUF) * NBUF

  def _wait_gather(b):
    pltpu.make_async_copy(g_hbm.at[sidx.at[0]], rows[b], sem_g).wait()

  def _wait_scatter(b):
    pltpu.make_async_copy(rows[b], acc.at[didx.at[0]], sem_s).wait()

  for b in range(GAHEAD):
    pltpu.async_copy(g_hbm.at[sidx.at[b]], rows[b], sem_g)

  @pl.loop(0, steps // NBUF)
  def _(t):
    for b in range(NBUF):
      m = t * NBUF + b
      _wait_gather(b)
      pltpu.async_copy(rows[b], acc.at[didx.at[m]], sem_s, add=True)
      @pl.when(m >= NBUF - GAHEAD)
      def _():
        _wait_scatter(b)
      @pl.when(m + GAHEAD < steps)
      def _():
        b2 = (b + GAHEAD) % NBUF
        pltpu.async_copy(g_hbm.at[sidx.at[m + GAHEAD]], rows[b2], sem_g)

  for m in range(main, steps):
    b = m % NBUF
    _wait_gather(b)
    pltpu.async_copy(rows[b], acc.at[didx.at[m]], sem_s, add=True)
    if m >= NBUF - GAHEAD:
      _wait_scatter(b)
    if m + GAHEAD < steps:
      pltpu.async_copy(g_hbm.at[sidx.at[m + GAHEAD]],
                       rows[(m + GAHEAD) % NBUF], sem_g)

  for _i in range(NBUF - GAHEAD):
    _wait_scatter(0)


@functools.cache
def _make_prop2_kernel():
  def body(glo_hbm, ghi_hbm, src_hbm, dst_hbm, out_hbm,
           sidx, didx, rows, acc, sem_g, sem_s):
    cid = lax.axis_index("c")
    sid = lax.axis_index("s")
    wid = sid * NC + cid
    base = sid * ROWS_PER_TILE

    nst = ROWS_W // K2
    pltpu.sync_copy(src_hbm.at[pl.ds(wid * nst, nst)], sidx)
    pltpu.sync_copy(dst_hbm.at[pl.ds(wid * nst, nst)], didx)

    for p, g_hbm in enumerate((glo_hbm, ghi_hbm)):
      _fill(rows[0], 0.0, WCH, DH)
      _zero_acc_slice(acc, rows[0].at[pl.ds(0, WCH)], base)
      plsc.subcore_barrier()
      _pipeline_phase(g_hbm, sidx, didx, rows, acc, sem_g, sem_s, K2)
      plsc.subcore_barrier()
      for kk in range(ROWS_PER_TILE // WCH):
        r0 = base + kk * WCH
        pltpu.sync_copy(acc.at[pl.ds(r0, WCH)], rows[0].at[pl.ds(0, WCH)])
        pltpu.async_copy(rows[0].at[pl.ds(0, WCH)],
                         out_hbm.at[cid, pl.ds(r0, WCH), pl.ds(p * DH, DH)],
                         sem_s).wait()
      if p == 0:
        plsc.subcore_barrier()

  return pl.kernel(
      body,
      out_type=jax.ShapeDtypeStruct((NC, NP, D_HID), jnp.float32),
      mesh=_mesh(),
      compiler_params=pltpu.CompilerParams(use_tc_tiling_on_sc=False),
      scratch_types=[
          pltpu.VMEM((ROWS_W // K2, K2 * CH), jnp.int32),
          pltpu.VMEM((ROWS_W // K2, K2 * CH), jnp.int32),
          [pltpu.VMEM((K2 * CH, DH), jnp.float32)] * NBUF,
          pltpu.VMEM_SHARED((NP, DH), jnp.float32),
          pltpu.SemaphoreType.DMA,
          pltpu.SemaphoreType.DMA,
      ],
  )


@functools.cache
def _make_prop16_kernel():
  def body(g_hbm, src_hbm, dst_hbm, out_hbm,
           sidx, didx, rows, acc, sem_g, sem_s):
    cid = lax.axis_index("c")
    sid = lax.axis_index("s")
    wid = sid * NC + cid
    base = sid * ROWS_PER_TILE

    nst = ROWS_W // K16
    pltpu.sync_copy(src_hbm.at[pl.ds(wid * nst, nst)], sidx)
    pltpu.sync_copy(dst_hbm.at[pl.ds(wid * nst, nst)], didx)
    _fill(rows[0], 0.0, WCH, DO)
    _zero_acc_slice(acc, rows[0].at[pl.ds(0, WCH)], base)
    plsc.subcore_barrier()
    _pipeline_phase(g_hbm, sidx, didx, rows, acc, sem_g, sem_s, K16)
    plsc.subcore_barrier()
    _writeout(acc, out_hbm.at[cid], rows[0].at[pl.ds(0, WCH)], sem_s, base)

  return pl.kernel(
      body,
      out_type=jax.ShapeDtypeStruct((NC, NP, DO), jnp.float32),
      mesh=_mesh(),
      compiler_params=pltpu.CompilerParams(use_tc_tiling_on_sc=False),
      scratch_types=[
          pltpu.VMEM((ROWS_W // K16, K16 * CH), jnp.int32),
          pltpu.VMEM((ROWS_W // K16, K16 * CH), jnp.int32),
          [pltpu.VMEM((K16 * CH, DO), jnp.float32)] * NBUF,
          pltpu.VMEM_SHARED((NP, DO), jnp.float32),
          pltpu.SemaphoreType.DMA,
          pltpu.SemaphoreType.DMA,
      ],
  )


_BR = 1024


def _tc_first(degp, x, w):
  def body(dp_ref, x_ref, w_ref, dinv_ref, glo_ref, ghi_ref):
    dv = lax.rsqrt(dp_ref[0, :, 0:1] + dp_ref[1, :, 0:1] + 1.0)
    dinv_ref[...] = dv
    g = dv * jnp.dot(x_ref[...], w_ref[...],
                     preferred_element_type=jnp.float32)
    glo_ref[...] = g[:, :DH]
    ghi_ref[...] = g[:, DH:]

  return pl.pallas_call(
      body,
      grid=(NP // _BR,),
      in_specs=[
          pl.BlockSpec((NC, _BR, DO), lambda i: (0, i, 0)),
          pl.BlockSpec((_BR, D_HID), lambda i: (i, 0)),
          pl.BlockSpec((D_HID, D_HID), lambda i: (0, 0)),
      ],
      out_specs=[
          pl.BlockSpec((_BR, 1), lambda i: (i, 0)),
          pl.BlockSpec((_BR, DH), lambda i: (i, 0)),
          pl.BlockSpec((_BR, DH), lambda i: (i, 0)),
      ],
      out_shape=[
          jax.ShapeDtypeStruct((NP, 1), jnp.float32),
          jax.ShapeDtypeStruct((NP, DH), jnp.float32),
          jax.ShapeDtypeStruct((NP, DH), jnp.float32),
      ],
  )(degp, x, w)


def _tc_combine(dinv, parts, glo, ghi, b, w, split_out):
  width_out = D_HID if split_out else DO

  def body(dv_ref, p_ref, glo_ref, ghi_ref, b_ref, w_ref, *out_refs):
    dv = dv_ref[...]
    h = (p_ref[0] + p_ref[1]
         + jnp.concatenate([glo_ref[...], ghi_ref[...]], axis=1))
    y = jnp.maximum(dv * h + b_ref[...], 0.0)
    g = dv * jnp.dot(y, w_ref[...], preferred_element_type=jnp.float32)
    if split_out:
      out_refs[0][...] = g[:, :DH]
      out_refs[1][...] = g[:, DH:]
    else:
      out_refs[0][...] = g

  if split_out:
    out_specs = [pl.BlockSpec((_BR, DH), lambda i: (i, 0))] * 2
    out_shape = [jax.ShapeDtypeStruct((NP, DH), jnp.float32)] * 2
  else:
    out_specs = [pl.BlockSpec((_BR, DO), lambda i: (i, 0))]
    out_shape = [jax.ShapeDtypeStruct((NP, DO), jnp.float32)]

  return pl.pallas_call(
      body,
      grid=(NP // _BR,),
      in_specs=[
          pl.BlockSpec((_BR, 1), lambda i: (i, 0)),
          pl.BlockSpec((NC, _BR, D_HID), lambda i: (0, i, 0)),
          pl.BlockSpec((_BR, DH), lambda i: (i, 0)),
          pl.BlockSpec((_BR, DH), lambda i: (i, 0)),
          pl.BlockSpec((1, D_HID), lambda i: (0, 0)),
          pl.BlockSpec((D_HID, width_out), lambda i: (0, 0)),
      ],
      out_specs=out_specs,
      out_shape=out_shape,
  )(dinv, parts, glo, ghi, b, w)


def _tc_final(dinv, parts, g, b, d_out):
  brf = 1000

  def body(dv_ref, p_ref, g_ref, b_ref, out_ref):
    v = dv_ref[...] * (p_ref[0] + p_ref[1] + g_ref[...]) + b_ref[...]
    out_ref[...] = v[:, :d_out]

  return pl.pallas_call(
      body,
      grid=(N // brf,),
      in_specs=[
          pl.BlockSpec((brf, 1), lambda i: (i, 0)),
          pl.BlockSpec((NC, brf, DO), lambda i: (0, i, 0)),
          pl.BlockSpec((brf, DO), lambda i: (i, 0)),
          pl.BlockSpec((1, DO), lambda i: (0, 0)),
      ],
      out_specs=pl.BlockSpec((brf, d_out), lambda i: (i, 0)),
      out_shape=jax.ShapeDtypeStruct((N, d_out), jnp.float32),
  )(dinv, parts, g, b)


@jax.jit
def kernel(x, edge_index, W1, b1, W2, b2, W3, b3):
  _deg_kernel = _make_deg_kernel()
  _prop2 = _make_prop2_kernel()
  _prop16 = _make_prop16_kernel()

  pad = (N + (jnp.arange(E_PAD - E, dtype=jnp.int32) % (NP - N))).astype(
      jnp.int32)
  src_flat = jnp.concatenate([edge_index[0], pad])
  dst_flat = jnp.concatenate([edge_index[1], pad])
  src2 = src_flat.reshape(NW * (ROWS_W // K2), K2 * CH)
  dst2 = dst_flat.reshape(NW * (ROWS_W // K2), K2 * CH)
  src16 = src_flat.reshape(NW * (ROWS_W // K16), K16 * CH)
  dst16 = dst_flat.reshape(NW * (ROWS_W // K16), K16 * CH)

  x_pad = jnp.zeros((NP, D_HID), jnp.float32).at[:N, :D_IN].set(x)
  w1p = jnp.zeros((D_HID, D_HID), jnp.float32).at[:D_IN].set(W1)
  w3p = jnp.zeros((D_HID, DO), jnp.float32).at[:, : W3.shape[1]].set(W3)
  b1r = b1.reshape(1, D_HID)
  b2r = b2.reshape(1, D_HID)
  b3r = jnp.zeros((1, DO), jnp.float32).at[0, : b3.shape[0]].set(b3)

  degp = _deg_kernel(dst16)
  dinv, g1lo, g1hi = _tc_first(degp, x_pad, w1p)
  p1 = _prop2(g1lo, g1hi, src2, dst2)
  g2lo, g2hi = _tc_combine(dinv, p1, g1lo, g1hi, b1r, W2, split_out=True)
  p2 = _prop2(g2lo, g2hi, src2, dst2)
  (g3,) = _tc_combine(dinv, p2, g2lo, g2hi, b2r, w3p, split_out=False)
  p3 = _prop16(g3, src16, dst16)
  return _tc_final(dinv, p3, g3, b3r, W3.shape[1])


# speedup vs baseline: 1.1502x; 1.0284x over previous
import functools

import jax
import jax.numpy as jnp
from jax import lax
from jax.experimental import pallas as pl
from jax.experimental.pallas import tpu as pltpu
from jax.experimental.pallas import tpu_sc as plsc

N = 10000
E = 640000
NP = 10240
D_IN = 116
D_HID = 128
DH = 64
DO = 16

NC = 2
NS = 16
NW = NC * NS
CH = 120
ROWS_W = 168
E_PAD = NW * ROWS_W * CH
ROWS_PER_TILE = NP // NS
WCH = 128
K2 = 2
K16 = 8
KDEG = 8
NBUF = 3
GAHEAD = 2
DEGWIN = 4


@functools.cache
def _mesh():
  return plsc.VectorSubcoreMesh(
      core_axis_name="c", subcore_axis_name="s", num_cores=NC, num_subcores=NS
  )


def _fill(ref, value, nrows, ncols):
  @pl.loop(0, nrows)
  def _(r):
    for j in range(ncols // 16):
      ref[r, pl.ds(j * 16, 16)] = jnp.full((16,), value, jnp.float32)


def _zero_acc_slice(acc, zbuf, base):
  for k in range(ROWS_PER_TILE // WCH):
    pltpu.sync_copy(zbuf, acc.at[pl.ds(base + k * WCH, WCH)])


def _writeout(acc, out_ref, rows, sem, base):
  for k in range(ROWS_PER_TILE // WCH):
    r0 = base + k * WCH
    pltpu.sync_copy(acc.at[pl.ds(r0, WCH)], rows)
    pltpu.async_copy(rows, out_ref.at[pl.ds(r0, WCH), pl.ds(0, DO)],
                     sem).wait()


@functools.cache
def _make_deg_kernel():
  steps = ROWS_W // KDEG

  def body(dst_hbm, out_hbm, didx, ones, rows, acc, sem_s):
    cid = lax.axis_index("c")
    sid = lax.axis_index("s")
    wid = sid * NC + cid
    base = sid * ROWS_PER_TILE

    pltpu.sync_copy(dst_hbm.at[pl.ds(wid * steps, steps)], didx)
    _fill(rows, 0.0, WCH, DO)
    _zero_acc_slice(acc, rows, base)
    _fill(ones, 1.0, KDEG * CH, DO)
    plsc.subcore_barrier()

    for m in range(DEGWIN):
      pltpu.async_copy(ones, acc.at[didx.at[m]], sem_s, add=True)

    @pl.loop(0, steps - DEGWIN)
    def _(m):
      pltpu.make_async_copy(ones, acc.at[didx.at[0]], sem_s).wait()
      pltpu.async_copy(ones, acc.at[didx.at[m + DEGWIN]], sem_s, add=True)

    for m in range(DEGWIN):
      pltpu.make_async_copy(ones, acc.at[didx.at[0]], sem_s).wait()

    plsc.subcore_barrier()
    _writeout(acc, out_hbm.at[cid], rows, sem_s, base)

  return pl.kernel(
      body,
      out_type=jax.ShapeDtypeStruct((NC, NP, D_HID), jnp.float32),
      mesh=_mesh(),
      compiler_params=pltpu.CompilerParams(use_tc_tiling_on_sc=False),
      scratch_types=[
          pltpu.VMEM((ROWS_W // KDEG, KDEG * CH), jnp.int32),
          pltpu.VMEM((KDEG * CH, DO), jnp.float32),
          pltpu.VMEM((WCH, DO), jnp.float32),
          pltpu.VMEM_SHARED((NP, DO), jnp.float32),
          pltpu.SemaphoreType.DMA,
      ],
  )


def _pipeline_phase(g_hbm, sidx, didx, rows, acc, sem_g, sem_s, k):
  steps = ROWS_W // k
  main = (steps // NBUF) * NBUF

  def _wait_gather(b):
    pltpu.make_async_copy(g_hbm.at[sidx.at[0]], rows[b], sem_g).wait()

  def _wait_scatter(b):
    pltpu.make_async_copy(rows[b], acc.at[didx.at[0]], sem_s).wait()

  for b in range(GAHEAD):
    pltpu.async_copy(g_hbm.at[sidx.at[b]], rows[b], sem_g)

  @pl.loop(0, steps // NBUF)
  def _(t):
    for b in range(NBUF):
      m = t * NBUF + b
      _wait_gather(b)
      pltpu.async_copy(rows[b], acc.at[didx.at[m]], sem_s, add=True)
      @pl.when(m >= NBUF - GAHEAD)
      def _():
        _wait_scatter(b)
      @pl.when(m + GAHEAD < steps)
      def _():
        b2 = (b + GAHEAD) % NBUF
        pltpu.async_copy(g_hbm.at[sidx.at[m + GAHEAD]], rows[b2], sem_g)

  for m in range(main, steps):
    b = m % NBUF
    _wait_gather(b)
    pltpu.async_copy(rows[b], acc.at[didx.at[m]], sem_s, add=True)
    if m >= NBUF - GAHEAD:
      _wait_scatter(b)
    if m + GAHEAD < steps:
      pltpu.async_copy(g_hbm.at[sidx.at[m + GAHEAD]],
                       rows[(m + GAHEAD) % NBUF], sem_g)

  for _i in range(NBUF - GAHEAD):
    _wait_scatter(0)


@functools.cache
def _make_prop2_kernel():
  def body(glo_hbm, ghi_hbm, src_hbm, dst_hbm, out_hbm,
           sidx, didx, rows, acc, sem_g, sem_s):
    cid = lax.axis_index("c")
    sid = lax.axis_index("s")
    wid = sid * NC + cid
    base = sid * ROWS_PER_TILE

    nst = ROWS_W // K2
    pltpu.sync_copy(src_hbm.at[pl.ds(wid * nst, nst)], sidx)
    pltpu.sync_copy(dst_hbm.at[pl.ds(wid * nst, nst)], didx)

    for p, g_hbm in enumerate((glo_hbm, ghi_hbm)):
      _fill(rows[0], 0.0, WCH, DH)
      _zero_acc_slice(acc, rows[0].at[pl.ds(0, WCH)], base)
      plsc.subcore_barrier()
      _pipeline_phase(g_hbm, sidx, didx, rows, acc, sem_g, sem_s, K2)
      plsc.subcore_barrier()
      for kk in range(ROWS_PER_TILE // WCH):
        r0 = base + kk * WCH
        pltpu.sync_copy(acc.at[pl.ds(r0, WCH)], rows[0].at[pl.ds(0, WCH)])
        pltpu.async_copy(rows[0].at[pl.ds(0, WCH)],
                         out_hbm.at[cid, pl.ds(r0, WCH), pl.ds(p * DH, DH)],
                         sem_s).wait()
      if p == 0:
        plsc.subcore_barrier()

  return pl.kernel(
      body,
      out_type=jax.ShapeDtypeStruct((NC, NP, D_HID), jnp.float32),
      mesh=_mesh(),
      compiler_params=pltpu.CompilerParams(use_tc_tiling_on_sc=False),
      scratch_types=[
          pltpu.VMEM((ROWS_W // K2, K2 * CH), jnp.int32),
          pltpu.VMEM((ROWS_W // K2, K2 * CH), jnp.int32),
          [pltpu.VMEM((K2 * CH, DH), jnp.float32)] * NBUF,
          pltpu.VMEM_SHARED((NP, DH), jnp.float32),
          pltpu.SemaphoreType.DMA,
          pltpu.SemaphoreType.DMA,
      ],
  )


@functools.cache
def _make_prop16_kernel():
  def body(g_hbm, src_hbm, dst_hbm, out_hbm,
           sidx, didx, rows, acc, sem_g, sem_s):
    cid = lax.axis_index("c")
    sid = lax.axis_index("s")
    wid = sid * NC + cid
    base = sid * ROWS_PER_TILE

    nst = ROWS_W // K16
    pltpu.sync_copy(src_hbm.at[pl.ds(wid * nst, nst)], sidx)
    pltpu.sync_copy(dst_hbm.at[pl.ds(wid * nst, nst)], didx)
    _fill(rows[0], 0.0, WCH, DO)
    _zero_acc_slice(acc, rows[0].at[pl.ds(0, WCH)], base)
    plsc.subcore_barrier()
    _pipeline_phase(g_hbm, sidx, didx, rows, acc, sem_g, sem_s, K16)
    plsc.subcore_barrier()
    _writeout(acc, out_hbm.at[cid], rows[0].at[pl.ds(0, WCH)], sem_s, base)

  return pl.kernel(
      body,
      out_type=jax.ShapeDtypeStruct((NC, NP, D_HID), jnp.float32),
      mesh=_mesh(),
      compiler_params=pltpu.CompilerParams(use_tc_tiling_on_sc=False),
      scratch_types=[
          pltpu.VMEM((ROWS_W // K16, K16 * CH), jnp.int32),
          pltpu.VMEM((ROWS_W // K16, K16 * CH), jnp.int32),
          [pltpu.VMEM((K16 * CH, DO), jnp.float32)] * NBUF,
          pltpu.VMEM_SHARED((NP, DO), jnp.float32),
          pltpu.SemaphoreType.DMA,
          pltpu.SemaphoreType.DMA,
      ],
  )


_BR = 1024


def _tc_first(degp, x, w):
  def body(dp_ref, x_ref, w_ref, dinv_ref, glo_ref, ghi_ref):
    dv = lax.rsqrt(dp_ref[0, :, 0:1] + dp_ref[1, :, 0:1] + 1.0)
    dinv_ref[...] = dv
    g = dv * jnp.dot(x_ref[...], w_ref[...],
                     preferred_element_type=jnp.float32)
    glo_ref[...] = g[:, :DH]
    ghi_ref[...] = g[:, DH:]

  return pl.pallas_call(
      body,
      grid=(NP // _BR,),
      in_specs=[
          pl.BlockSpec((NC, _BR, D_HID), lambda i: (0, i, 0)),
          pl.BlockSpec((_BR, D_HID), lambda i: (i, 0)),
          pl.BlockSpec((D_HID, D_HID), lambda i: (0, 0)),
      ],
      out_specs=[
          pl.BlockSpec((_BR, 1), lambda i: (i, 0)),
          pl.BlockSpec((_BR, DH), lambda i: (i, 0)),
          pl.BlockSpec((_BR, DH), lambda i: (i, 0)),
      ],
      out_shape=[
          jax.ShapeDtypeStruct((NP, 1), jnp.float32),
          jax.ShapeDtypeStruct((NP, DH), jnp.float32),
          jax.ShapeDtypeStruct((NP, DH), jnp.float32),
      ],
  )(degp, x, w)


def _tc_combine(dinv, parts, glo, ghi, b, w, split_out):
  width_out = D_HID if split_out else DO

  def body(dv_ref, p_ref, glo_ref, ghi_ref, b_ref, w_ref, *out_refs):
    dv = dv_ref[...]
    h = (p_ref[0] + p_ref[1]
         + jnp.concatenate([glo_ref[...], ghi_ref[...]], axis=1))
    y = jnp.maximum(dv * h + b_ref[...], 0.0)
    g = dv * jnp.dot(y, w_ref[...], preferred_element_type=jnp.float32)
    if split_out:
      out_refs[0][...] = g[:, :DH]
      out_refs[1][...] = g[:, DH:]
    else:
      out_refs[0][...] = g

  if split_out:
    out_specs = [pl.BlockSpec((_BR, DH), lambda i: (i, 0))] * 2
    out_shape = [jax.ShapeDtypeStruct((NP, DH), jnp.float32)] * 2
  else:
    out_specs = [pl.BlockSpec((_BR, DO), lambda i: (i, 0))]
    out_shape = [jax.ShapeDtypeStruct((NP, DO), jnp.float32)]

  return pl.pallas_call(
      body,
      grid=(NP // _BR,),
      in_specs=[
          pl.BlockSpec((_BR, 1), lambda i: (i, 0)),
          pl.BlockSpec((NC, _BR, D_HID), lambda i: (0, i, 0)),
          pl.BlockSpec((_BR, DH), lambda i: (i, 0)),
          pl.BlockSpec((_BR, DH), lambda i: (i, 0)),
          pl.BlockSpec((1, D_HID), lambda i: (0, 0)),
          pl.BlockSpec((D_HID, width_out), lambda i: (0, 0)),
      ],
      out_specs=out_specs,
      out_shape=out_shape,
  )(dinv, parts, glo, ghi, b, w)


def _tc_final(dinv, parts, g, b, d_out):
  brf = 1000

  def body(dv_ref, p_ref, g_ref, b_ref, out_ref):
    psum = p_ref[0, :, :DO] + p_ref[1, :, :DO]
    v = dv_ref[...] * (psum + g_ref[...]) + b_ref[...]
    out_ref[...] = v[:, :d_out]

  return pl.pallas_call(
      body,
      grid=(N // brf,),
      in_specs=[
          pl.BlockSpec((brf, 1), lambda i: (i, 0)),
          pl.BlockSpec((NC, brf, D_HID), lambda i: (0, i, 0)),
          pl.BlockSpec((brf, DO), lambda i: (i, 0)),
          pl.BlockSpec((1, DO), lambda i: (0, 0)),
      ],
      out_specs=pl.BlockSpec((brf, d_out), lambda i: (i, 0)),
      out_shape=jax.ShapeDtypeStruct((N, d_out), jnp.float32),
  )(dinv, parts, g, b)


@jax.jit
def kernel(x, edge_index, W1, b1, W2, b2, W3, b3):
  _deg_kernel = _make_deg_kernel()
  _prop2 = _make_prop2_kernel()
  _prop16 = _make_prop16_kernel()

  pad = (N + (jnp.arange(E_PAD - E, dtype=jnp.int32) % (NP - N))).astype(
      jnp.int32)
  src_flat = jnp.concatenate([edge_index[0], pad])
  dst_flat = jnp.concatenate([edge_index[1], pad])
  src2 = src_flat.reshape(NW * (ROWS_W // K2), K2 * CH)
  dst2 = dst_flat.reshape(NW * (ROWS_W // K2), K2 * CH)
  src16 = src_flat.reshape(NW * (ROWS_W // K16), K16 * CH)
  dst16 = dst_flat.reshape(NW * (ROWS_W // K16), K16 * CH)

  x_pad = jnp.zeros((NP, D_HID), jnp.float32).at[:N, :D_IN].set(x)
  w1p = jnp.zeros((D_HID, D_HID), jnp.float32).at[:D_IN].set(W1)
  w3p = jnp.zeros((D_HID, DO), jnp.float32).at[:, : W3.shape[1]].set(W3)
  b1r = b1.reshape(1, D_HID)
  b2r = b2.reshape(1, D_HID)
  b3r = jnp.zeros((1, DO), jnp.float32).at[0, : b3.shape[0]].set(b3)

  degp = _deg_kernel(dst16)
  dinv, g1lo, g1hi = _tc_first(degp, x_pad, w1p)
  p1 = _prop2(g1lo, g1hi, src2, dst2)
  g2lo, g2hi = _tc_combine(dinv, p1, g1lo, g1hi, b1r, W2, split_out=True)
  p2 = _prop2(g2lo, g2hi, src2, dst2)
  (g3,) = _tc_combine(dinv, p2, g2lo, g2hi, b2r, w3p, split_out=False)
  p3 = _prop16(g3, src16, dst16)
  return _tc_final(dinv, p3, g3, b3r, W3.shape[1])


# speedup vs baseline: 1.1531x; 1.0026x over previous
import functools

import jax
import jax.numpy as jnp
from jax import lax
from jax.experimental import pallas as pl
from jax.experimental.pallas import tpu as pltpu
from jax.experimental.pallas import tpu_sc as plsc

N = 10000
E = 640000
NP = 10240
D_IN = 116
D_HID = 128
DH = 64
DO = 16

NC = 2
NS = 16
NW = NC * NS
CH = 120
ROWS_W = 168
E_PAD = NW * ROWS_W * CH
ROWS_PER_TILE = NP // NS
WCH = 128
K2 = 2
K16 = 8
KDEG = 8
NBUF = 3
GAHEAD = 2
DEGWIN = 4


@functools.cache
def _mesh():
  return plsc.VectorSubcoreMesh(
      core_axis_name="c", subcore_axis_name="s", num_cores=NC, num_subcores=NS
  )


def _fill(ref, value, nrows, ncols):
  @pl.loop(0, nrows)
  def _(r):
    for j in range(ncols // 16):
      ref[r, pl.ds(j * 16, 16)] = jnp.full((16,), value, jnp.float32)


def _zero_acc_slice(acc, zbuf, base):
  for k in range(ROWS_PER_TILE // WCH):
    pltpu.sync_copy(zbuf, acc.at[pl.ds(base + k * WCH, WCH)])


def _writeout(acc, out_ref, rows, sem, base):
  for k in range(ROWS_PER_TILE // WCH):
    r0 = base + k * WCH
    pltpu.sync_copy(acc.at[pl.ds(r0, WCH)], rows)
    pltpu.async_copy(rows, out_ref.at[pl.ds(r0, WCH)], sem).wait()


@functools.cache
def _make_deg_kernel():
  steps = ROWS_W // KDEG

  def body(dst_hbm, out_hbm, didx, ones, rows, acc, sem_s):
    cid = lax.axis_index("c")
    sid = lax.axis_index("s")
    wid = sid * NC + cid
    base = sid * ROWS_PER_TILE

    pltpu.sync_copy(dst_hbm.at[pl.ds(wid * steps, steps)], didx)
    _fill(rows, 0.0, WCH, DO)
    _zero_acc_slice(acc, rows, base)
    _fill(ones, 1.0, KDEG * CH, DO)
    plsc.subcore_barrier()

    for m in range(DEGWIN):
      pltpu.async_copy(ones, acc.at[didx.at[m]], sem_s, add=True)

    @pl.loop(0, steps - DEGWIN)
    def _(m):
      pltpu.make_async_copy(ones, acc.at[didx.at[0]], sem_s).wait()
      pltpu.async_copy(ones, acc.at[didx.at[m + DEGWIN]], sem_s, add=True)

    for m in range(DEGWIN):
      pltpu.make_async_copy(ones, acc.at[didx.at[0]], sem_s).wait()

    plsc.subcore_barrier()
    _writeout(acc, out_hbm.at[:, pl.ds(cid * DO, DO)], rows, sem_s, base)

  return pl.kernel(
      body,
      out_type=jax.ShapeDtypeStruct((NP, D_HID), jnp.float32),
      mesh=_mesh(),
      compiler_params=pltpu.CompilerParams(use_tc_tiling_on_sc=False),
      scratch_types=[
          pltpu.VMEM((ROWS_W // KDEG, KDEG * CH), jnp.int32),
          pltpu.VMEM((KDEG * CH, DO), jnp.float32),
          pltpu.VMEM((WCH, DO), jnp.float32),
          pltpu.VMEM_SHARED((NP, DO), jnp.float32),
          pltpu.SemaphoreType.DMA,
      ],
  )


def _pipeline_phase(g_hbm, sidx, didx, rows, acc, sem_g, sem_s, k):
  steps = ROWS_W // k
  main = (steps // NBUF) * NBUF

  def _wait_gather(b):
    pltpu.make_async_copy(g_hbm.at[sidx.at[0]], rows[b], sem_g).wait()

  def _wait_scatter(b):
    pltpu.make_async_copy(rows[b], acc.at[didx.at[0]], sem_s).wait()

  for b in range(GAHEAD):
    pltpu.async_copy(g_hbm.at[sidx.at[b]], rows[b], sem_g)

  @pl.loop(0, steps // NBUF)
  def _(t):
    for b in range(NBUF):
      m = t * NBUF + b
      _wait_gather(b)
      pltpu.async_copy(rows[b], acc.at[didx.at[m]], sem_s, add=True)
      @pl.when(m >= NBUF - GAHEAD)
      def _():
        _wait_scatter(b)
      @pl.when(m + GAHEAD < steps)
      def _():
        b2 = (b + GAHEAD) % NBUF
        pltpu.async_copy(g_hbm.at[sidx.at[m + GAHEAD]], rows[b2], sem_g)

  for m in range(main, steps):
    b = m % NBUF
    _wait_gather(b)
    pltpu.async_copy(rows[b], acc.at[didx.at[m]], sem_s, add=True)
    if m >= NBUF - GAHEAD:
      _wait_scatter(b)
    if m + GAHEAD < steps:
      pltpu.async_copy(g_hbm.at[sidx.at[m + GAHEAD]],
                       rows[(m + GAHEAD) % NBUF], sem_g)

  for _i in range(NBUF - GAHEAD):
    _wait_scatter(0)


@functools.cache
def _make_prop2_kernel():
  def body(glo_hbm, ghi_hbm, src_hbm, dst_hbm, out_hbm,
           sidx, didx, rows, acc, sem_g, sem_s):
    cid = lax.axis_index("c")
    sid = lax.axis_index("s")
    wid = sid * NC + cid
    base = sid * ROWS_PER_TILE

    nst = ROWS_W // K2
    pltpu.sync_copy(src_hbm.at[pl.ds(wid * nst, nst)], sidx)
    pltpu.sync_copy(dst_hbm.at[pl.ds(wid * nst, nst)], didx)

    for p, g_hbm in enumerate((glo_hbm, ghi_hbm)):
      _fill(rows[0], 0.0, WCH, DH)
      _zero_acc_slice(acc, rows[0].at[pl.ds(0, WCH)], base)
      plsc.subcore_barrier()
      _pipeline_phase(g_hbm, sidx, didx, rows, acc, sem_g, sem_s, K2)
      plsc.subcore_barrier()
      for kk in range(ROWS_PER_TILE // WCH):
        r0 = base + kk * WCH
        pltpu.sync_copy(acc.at[pl.ds(r0, WCH)], rows[0].at[pl.ds(0, WCH)])
        pltpu.async_copy(rows[0].at[pl.ds(0, WCH)],
                         out_hbm.at[cid, pl.ds(r0, WCH), pl.ds(p * DH, DH)],
                         sem_s).wait()
      if p == 0:
        plsc.subcore_barrier()

  return pl.kernel(
      body,
      out_type=jax.ShapeDtypeStruct((NC, NP, D_HID), jnp.float32),
      mesh=_mesh(),
      compiler_params=pltpu.CompilerParams(use_tc_tiling_on_sc=False),
      scratch_types=[
          pltpu.VMEM((ROWS_W // K2, K2 * CH), jnp.int32),
          pltpu.VMEM((ROWS_W // K2, K2 * CH), jnp.int32),
          [pltpu.VMEM((K2 * CH, DH), jnp.float32)] * NBUF,
          pltpu.VMEM_SHARED((NP, DH), jnp.float32),
          pltpu.SemaphoreType.DMA,
          pltpu.SemaphoreType.DMA,
      ],
  )


@functools.cache
def _make_prop16_kernel():
  def body(g_hbm, src_hbm, dst_hbm, out_hbm,
           sidx, didx, rows, acc, sem_g, sem_s):
    cid = lax.axis_index("c")
    sid = lax.axis_index("s")
    wid = sid * NC + cid
    base = sid * ROWS_PER_TILE

    nst = ROWS_W // K16
    pltpu.sync_copy(src_hbm.at[pl.ds(wid * nst, nst)], sidx)
    pltpu.sync_copy(dst_hbm.at[pl.ds(wid * nst, nst)], didx)
    _fill(rows[0], 0.0, WCH, DO)
    _zero_acc_slice(acc, rows[0].at[pl.ds(0, WCH)], base)
    plsc.subcore_barrier()
    _pipeline_phase(g_hbm, sidx, didx, rows, acc, sem_g, sem_s, K16)
    plsc.subcore_barrier()
    _writeout(acc, out_hbm.at[:, pl.ds(cid * DO, DO)],
              rows[0].at[pl.ds(0, WCH)], sem_s, base)

  return pl.kernel(
      body,
      out_type=jax.ShapeDtypeStruct((NP, D_HID), jnp.float32),
      mesh=_mesh(),
      compiler_params=pltpu.CompilerParams(use_tc_tiling_on_sc=False),
      scratch_types=[
          pltpu.VMEM((ROWS_W // K16, K16 * CH), jnp.int32),
          pltpu.VMEM((ROWS_W // K16, K16 * CH), jnp.int32),
          [pltpu.VMEM((K16 * CH, DO), jnp.float32)] * NBUF,
          pltpu.VMEM_SHARED((NP, DO), jnp.float32),
          pltpu.SemaphoreType.DMA,
          pltpu.SemaphoreType.DMA,
      ],
  )


_BR = 1024


def _tc_first(degp, x, w):
  def body(dp_ref, x_ref, w_ref, dinv_ref, glo_ref, ghi_ref):
    dv = lax.rsqrt(dp_ref[:, 0:1] + dp_ref[:, DO:DO + 1] + 1.0)
    dinv_ref[...] = dv
    g = dv * jnp.dot(x_ref[...], w_ref[...],
                     preferred_element_type=jnp.float32)
    glo_ref[...] = g[:, :DH]
    ghi_ref[...] = g[:, DH:]

  return pl.pallas_call(
      body,
      grid=(NP // _BR,),
      in_specs=[
          pl.BlockSpec((_BR, D_HID), lambda i: (i, 0)),
          pl.BlockSpec((_BR, D_HID), lambda i: (i, 0)),
          pl.BlockSpec((D_HID, D_HID), lambda i: (0, 0)),
      ],
      out_specs=[
          pl.BlockSpec((_BR, 1), lambda i: (i, 0)),
          pl.BlockSpec((_BR, DH), lambda i: (i, 0)),
          pl.BlockSpec((_BR, DH), lambda i: (i, 0)),
      ],
      out_shape=[
          jax.ShapeDtypeStruct((NP, 1), jnp.float32),
          jax.ShapeDtypeStruct((NP, DH), jnp.float32),
          jax.ShapeDtypeStruct((NP, DH), jnp.float32),
      ],
  )(degp, x, w)


def _tc_combine(dinv, parts, glo, ghi, b, w, split_out):
  width_out = D_HID if split_out else DO

  def body(dv_ref, p_ref, glo_ref, ghi_ref, b_ref, w_ref, *out_refs):
    dv = dv_ref[...]
    h = (p_ref[0] + p_ref[1]
         + jnp.concatenate([glo_ref[...], ghi_ref[...]], axis=1))
    y = jnp.maximum(dv * h + b_ref[...], 0.0)
    g = dv * jnp.dot(y, w_ref[...], preferred_element_type=jnp.float32)
    if split_out:
      out_refs[0][...] = g[:, :DH]
      out_refs[1][...] = g[:, DH:]
    else:
      out_refs[0][...] = g

  if split_out:
    out_specs = [pl.BlockSpec((_BR, DH), lambda i: (i, 0))] * 2
    out_shape = [jax.ShapeDtypeStruct((NP, DH), jnp.float32)] * 2
  else:
    out_specs = [pl.BlockSpec((_BR, DO), lambda i: (i, 0))]
    out_shape = [jax.ShapeDtypeStruct((NP, DO), jnp.float32)]

  return pl.pallas_call(
      body,
      grid=(NP // _BR,),
      in_specs=[
          pl.BlockSpec((_BR, 1), lambda i: (i, 0)),
          pl.BlockSpec((NC, _BR, D_HID), lambda i: (0, i, 0)),
          pl.BlockSpec((_BR, DH), lambda i: (i, 0)),
          pl.BlockSpec((_BR, DH), lambda i: (i, 0)),
          pl.BlockSpec((1, D_HID), lambda i: (0, 0)),
          pl.BlockSpec((D_HID, width_out), lambda i: (0, 0)),
      ],
      out_specs=out_specs,
      out_shape=out_shape,
  )(dinv, parts, glo, ghi, b, w)


def _tc_final(dinv, parts, g, b, d_out):
  brf = 1000

  def body(dv_ref, p_ref, g_ref, b_ref, out_ref):
    psum = p_ref[:, :DO] + p_ref[:, DO:2 * DO]
    v = dv_ref[...] * (psum + g_ref[...]) + b_ref[...]
    out_ref[...] = v[:, :d_out]

  return pl.pallas_call(
      body,
      grid=(N // brf,),
      in_specs=[
          pl.BlockSpec((brf, 1), lambda i: (i, 0)),
          pl.BlockSpec((brf, D_HID), lambda i: (i, 0)),
          pl.BlockSpec((brf, DO), lambda i: (i, 0)),
          pl.BlockSpec((1, DO), lambda i: (0, 0)),
      ],
      out_specs=pl.BlockSpec((brf, d_out), lambda i: (i, 0)),
      out_shape=jax.ShapeDtypeStruct((N, d_out), jnp.float32),
  )(dinv, parts, g, b)


@jax.jit
def kernel(x, edge_index, W1, b1, W2, b2, W3, b3):
  _deg_kernel = _make_deg_kernel()
  _prop2 = _make_prop2_kernel()
  _prop16 = _make_prop16_kernel()

  pad = (N + (jnp.arange(E_PAD - E, dtype=jnp.int32) % (NP - N))).astype(
      jnp.int32)
  src_flat = jnp.concatenate([edge_index[0], pad])
  dst_flat = jnp.concatenate([edge_index[1], pad])
  src2 = src_flat.reshape(NW * (ROWS_W // K2), K2 * CH)
  dst2 = dst_flat.reshape(NW * (ROWS_W // K2), K2 * CH)
  src16 = src_flat.reshape(NW * (ROWS_W // K16), K16 * CH)
  dst16 = dst_flat.reshape(NW * (ROWS_W // K16), K16 * CH)

  x_pad = jnp.zeros((NP, D_HID), jnp.float32).at[:N, :D_IN].set(x)
  w1p = jnp.zeros((D_HID, D_HID), jnp.float32).at[:D_IN].set(W1)
  w3p = jnp.zeros((D_HID, DO), jnp.float32).at[:, : W3.shape[1]].set(W3)
  b1r = b1.reshape(1, D_HID)
  b2r = b2.reshape(1, D_HID)
  b3r = jnp.zeros((1, DO), jnp.float32).at[0, : b3.shape[0]].set(b3)

  degp = _deg_kernel(dst16)
  dinv, g1lo, g1hi = _tc_first(degp, x_pad, w1p)
  p1 = _prop2(g1lo, g1hi, src2, dst2)
  g2lo, g2hi = _tc_combine(dinv, p1, g1lo, g1hi, b1r, W2, split_out=True)
  p2 = _prop2(g2lo, g2hi, src2, dst2)
  (g3,) = _tc_combine(dinv, p2, g2lo, g2hi, b2r, w3p, split_out=False)
  p3 = _prop16(g3, src16, dst16)
  return _tc_final(dinv, p3, g3, b3r, W3.shape[1])
